# baseline (speedup 1.0000x reference)
import jax
import jax.numpy as jnp
from jax import lax
from jax.experimental import pallas as pl
from jax.experimental.pallas import tpu as pltpu
from jax.experimental.pallas import tpu_sc as plsc

N = 10000
E = 320000
NFEAT = 128
NHID = 64
OUTD2 = 64
NHEADS = 4
ALPHA = 0.2

NC = 2
NS = 16
LANES = 16
FEAT = 64

NP = 10240
CH = 128
EPTOT = 331776
ROWS_PT = NP // NS
ZR = 128

_BN = 1000


def _leaky(v):
    return jnp.where(v >= 0.0, v, ALPHA * v)


def _elu(v):
    return jnp.where(v > 0.0, v, jnp.exp(v) - 1.0)


def _tc1_body(x_ref, w_ref, b_ref, as_ref, ad_ref,
              h0_ref, h1_ref, h2_ref, h3_ref, es_ref, ed_ref, mo_ref, mx_s):
    i = pl.program_id(0)
    h = jnp.dot(x_ref[...], w_ref[...], preferred_element_type=jnp.float32) + b_ref[...]
    for j, href in enumerate([h0_ref, h1_ref, h2_ref, h3_ref]):
        href[...] = h[:, j * NHID:(j + 1) * NHID]
    es = jnp.dot(h, as_ref[...], preferred_element_type=jnp.float32)
    ed = jnp.dot(h, ad_ref[...], preferred_element_type=jnp.float32)
    es_ref[...] = es
    ed_ref[...] = ed
    bm = jnp.broadcast_to(
        jnp.concatenate([es.max(axis=0), ed.max(axis=0)])[:, None], (8, LANES))

    @pl.when(i == 0)
    def _():
        mx_s[...] = bm

    @pl.when(i > 0)
    def _():
        mx_s[...] = jnp.maximum(mx_s[...], bm)

    m = mx_s[...]
    mo_ref[...] = _leaky(m[:4, :] + m[4:, :])


def _tc1(x, w1cat, b1cat, a1s, a1d):
    grid = (N // _BN,)
    return pl.pallas_call(
        _tc1_body,
        grid=grid,
        in_specs=[
            pl.BlockSpec((_BN, NFEAT), lambda i: (i, 0)),
            pl.BlockSpec((NFEAT, 2 * NFEAT), lambda i: (0, 0)),
            pl.BlockSpec((1, 2 * NFEAT), lambda i: (0, 0)),
            pl.BlockSpec((2 * NFEAT, NHEADS), lambda i: (0, 0)),
            pl.BlockSpec((2 * NFEAT, NHEADS), lambda i: (0, 0)),
        ],
        out_specs=[
            pl.BlockSpec((_BN, NHID), lambda i: (i, 0)),
            pl.BlockSpec((_BN, NHID), lambda i: (i, 0)),
            pl.BlockSpec((_BN, NHID), lambda i: (i, 0)),
            pl.BlockSpec((_BN, NHID), lambda i: (i, 0)),
            pl.BlockSpec((_BN, NHEADS), lambda i: (i, 0)),
            pl.BlockSpec((_BN, NHEADS), lambda i: (i, 0)),
            pl.BlockSpec((NHEADS, LANES), lambda i: (0, 0)),
        ],
        out_shape=[
            jax.ShapeDtypeStruct((N, NHID), jnp.float32),
            jax.ShapeDtypeStruct((N, NHID), jnp.float32),
            jax.ShapeDtypeStruct((N, NHID), jnp.float32),
            jax.ShapeDtypeStruct((N, NHID), jnp.float32),
            jax.ShapeDtypeStruct((N, NHEADS), jnp.float32),
            jax.ShapeDtypeStruct((N, NHEADS), jnp.float32),
            jax.ShapeDtypeStruct((NHEADS, LANES), jnp.float32),
        ],
        scratch_shapes=[pltpu.VMEM((8, LANES), jnp.float32)],
    )(x, w1cat, b1cat, a1s, a1d)


def _make_sc_body(ept, cb):
    nchunk = ept // CH
    assert nchunk % 3 == 0 and nchunk >= 6

    def body(h_a, h_b, es_a, es_b, ed_a, ed_b, mo, src, dst,
             num_out, den_out,
             es_v, ed_v, mo_v, den_v,
             exv0, exv1, exv2, srcv0, srcv1, srcv2, dstv0, dstv1, dstv2,
             dsc0, dsc1, dsc2, rows0, rows1, rows2, zbuf, num_sh,
             isem0, isem1, isem2, gsem0, gsem1, gsem2, ssem0, ssem1, ssem2):
        c = lax.axis_index("c")
        s = lax.axis_index("s")

        @pl.when(c == 0)
        def _():
            pltpu.sync_copy(es_a, es_v)
            pltpu.sync_copy(ed_a, ed_v)

        @pl.when(c == 1)
        def _():
            pltpu.sync_copy(es_b, es_v)
            pltpu.sync_copy(ed_b, ed_v)

        pltpu.sync_copy(mo, mo_v)

        def zden(k, carry):
            den_v[pl.ds(k * LANES, LANES)] = jnp.zeros((LANES,), jnp.float32)
            return carry

        lax.fori_loop(0, NP // LANES, zden, 0)

        def zz(k, carry):
            z = jnp.zeros((LANES,), jnp.float32)
            for j in range(FEAT // LANES):
                zbuf[k, pl.ds(j * LANES, LANES)] = z
            return carry

        lax.fori_loop(0, ZR, zz, 0)
        rbase = pl.multiple_of(s * ROWS_PT, 8)
        for t in range(ROWS_PT // ZR):
            pltpu.sync_copy(zbuf, num_sh.at[pl.ds(rbase + t * ZR, ZR)])
        plsc.subcore_barrier()

        bufs = ((srcv0, dstv0, dsc0, rows0, exv0, isem0, gsem0, ssem0),
                (srcv1, dstv1, dsc1, rows1, exv1, isem1, gsem1, ssem1),
                (srcv2, dstv2, dsc2, rows2, exv2, isem2, gsem2, ssem2))

        def edge_loop(h_ref):
            moff = mo_v[pl.ds(pl.multiple_of(c * LANES, 8), LANES)]

            def off(b):
                o = c * cb + s * ept + b * CH
                o = jnp.minimum(o, EPTOT - CH)
                return pl.multiple_of(o, 8)

            def issue_idx(b, buf):
                srcv, dstv, isem = bufs[buf][0], bufs[buf][1], bufs[buf][5]
                o = off(b)
                pltpu.async_copy(src.at[pl.ds(o, CH)], srcv, isem)
                pltpu.async_copy(dst.at[pl.ds(o, CH)], dstv, isem)

            def wait_idx(buf):
                srcv, dstv, isem = bufs[buf][0], bufs[buf][1], bufs[buf][5]
                pltpu.make_async_copy(src.at[pl.ds(0, CH)], srcv, isem).wait()
                pltpu.make_async_copy(dst.at[pl.ds(0, CH)], dstv, isem).wait()

            def issue_gather(buf):
                srcv, rows, gsem = bufs[buf][0], bufs[buf][3], bufs[buf][6]
                pltpu.async_copy(h_ref.at[srcv], rows, gsem)

            def wait_gather(buf):
                rows, gsem = bufs[buf][3], bufs[buf][6]
                pltpu.make_async_copy(h_ref.at[pl.ds(0, CH)], rows, gsem).wait()

            def wait_scatter(buf):
                rows, ssem = bufs[buf][3], bufs[buf][7]
                pltpu.make_async_copy(h_ref.at[pl.ds(0, CH)], rows, ssem).wait()

            def process(b, buf, skip_ws):
                srcv, dstv, dsc, rows, exv, isem, gsem, ssem = bufs[buf]
                nxt = (buf + 1) % 3
                for k in range(CH // LANES):
                    sv = srcv[pl.ds(k * LANES, LANES)]
                    dv = dstv[pl.ds(k * LANES, LANES)]
                    a = (plsc.load_gather(es_v, [sv])
                         + plsc.load_gather(ed_v, [dv]))
                    ex = jnp.exp(_leaky(a) - moff)
                    exv[pl.ds(k * LANES, LANES)] = ex
                    plsc.addupdate_scatter(den_v, [dv], ex)
                    dsc[pl.ds(k * LANES, LANES)] = dv
                if not skip_ws:
                    wait_scatter(nxt)
                wait_idx(nxt)
                issue_gather(nxt)
                wait_gather(buf)
                issue_idx(b + 3, buf)
                def scale(i2, carry2):
                    for u in range(4):
                        i = i2 * 4 + u
                        bidx = jnp.zeros((LANES,), jnp.int32) + i
                        b0 = plsc.load_gather(exv, [bidx])
                        for j in range(FEAT // LANES):
                            sl = rows[i, pl.ds(j * LANES, LANES)]
                            rows[i, pl.ds(j * LANES, LANES)] = sl * b0
                    return carry2

                lax.fori_loop(0, CH // 4, scale, 0)
                pltpu.async_copy(rows, num_sh.at[dsc], ssem, add=True)

            issue_idx(0, 0)
            issue_idx(1, 1)
            issue_idx(2, 2)
            wait_idx(0)
            issue_gather(0)
            process(0, 0, True)
            process(1, 1, True)
            process(2, 2, False)

            def body3(k, carry):
                process(3 * k, 0, False)
                process(3 * k + 1, 1, False)
                process(3 * k + 2, 2, False)
                return carry

            lax.fori_loop(1, nchunk // 3, body3, 0)
            wait_idx((nchunk + 1) % 3)
            wait_idx((nchunk + 2) % 3)
            wait_gather(nchunk % 3)
            wait_scatter((nchunk - 2) % 3)
            wait_scatter((nchunk - 1) % 3)

        @pl.when(c == 0)
        def _():
            edge_loop(h_a)

        @pl.when(c == 1)
        def _():
            edge_loop(h_b)

        plsc.subcore_barrier()
        pltpu.sync_copy(num_sh.at[pl.ds(rbase, ROWS_PT)],
                        num_out.at[c].at[pl.ds(rbase, ROWS_PT)])
        doff = pl.multiple_of((c * NS + s) * NP, 8)
        pltpu.sync_copy(den_v, den_out.at[pl.ds(doff, NP)])

    return body


def _sc_call(ept, cb, h_a, h_b, es_a, es_b, ed_a, ed_b, mo, src, dst):
    mesh = plsc.VectorSubcoreMesh(core_axis_name="c", subcore_axis_name="s",
                                  num_cores=NC, num_subcores=NS)
    f = pl.kernel(
        _make_sc_body(ept, cb),
        out_type=[
            jax.ShapeDtypeStruct((NC, NP, FEAT), jnp.float32),
            jax.ShapeDtypeStruct((NC * NS * NP,), jnp.float32),
        ],
        mesh=mesh,
        compiler_params=pltpu.CompilerParams(
            needs_layout_passes=False, use_tc_tiling_on_sc=False),
        scratch_types=[
            pltpu.VMEM((NP,), jnp.float32),
            pltpu.VMEM((NP,), jnp.float32),
            pltpu.VMEM((NC * LANES,), jnp.float32),
            pltpu.VMEM((NP,), jnp.float32),
            pltpu.VMEM((CH,), jnp.float32),
            pltpu.VMEM((CH,), jnp.float32),
            pltpu.VMEM((CH,), jnp.float32),
            pltpu.VMEM((CH,), jnp.int32),
            pltpu.VMEM((CH,), jnp.int32),
            pltpu.VMEM((CH,), jnp.int32),
            pltpu.VMEM((CH,), jnp.int32),
            pltpu.VMEM((CH,), jnp.int32),
            pltpu.VMEM((CH,), jnp.int32),
            pltpu.VMEM((CH,), jnp.int32),
            pltpu.VMEM((CH,), jnp.int32),
            pltpu.VMEM((CH,), jnp.int32),
            pltpu.VMEM((CH, FEAT), jnp.float32),
            pltpu.VMEM((CH, FEAT), jnp.float32),
            pltpu.VMEM((CH, FEAT), jnp.float32),
            pltpu.VMEM((ZR, FEAT), jnp.float32),
            pltpu.VMEM_SHARED((NP, FEAT), jnp.float32),
            pltpu.SemaphoreType.DMA,
            pltpu.SemaphoreType.DMA,
            pltpu.SemaphoreType.DMA,
            pltpu.SemaphoreType.DMA,
            pltpu.SemaphoreType.DMA,
            pltpu.SemaphoreType.DMA,
            pltpu.SemaphoreType.DMA,
            pltpu.SemaphoreType.DMA,
            pltpu.SemaphoreType.DMA,
        ],
    )
    return f(h_a, h_b, es_a, es_b, ed_a, ed_b, mo, src, dst)


def _tc2_body(numa_ref, numb_ref, den_ref, w2_ref, b2_ref, a2s_ref, a2d_ref,
              h2_ref, es2_ref, ed2_ref, mo2_ref, mx_s):
    i = pl.program_id(0)
    acc = jnp.zeros((_BN, OUTD2), jnp.float32)
    for h in range(NHEADS):
        nh = (numa_ref if h < 2 else numb_ref)[h % 2]
        dh = jnp.sum(den_ref[:, h * NS:(h + 1) * NS], axis=1)
        o = _elu(nh / (dh[:, None] + 1e-16))
        acc = acc + jnp.dot(o, w2_ref[h * NHID:(h + 1) * NHID, :],
                            preferred_element_type=jnp.float32)
    h2 = acc + b2_ref[...]
    h2_ref[...] = h2
    es2 = jnp.dot(h2, a2s_ref[...], preferred_element_type=jnp.float32)
    ed2 = jnp.dot(h2, a2d_ref[...], preferred_element_type=jnp.float32)
    es2_ref[...] = es2
    ed2_ref[...] = ed2
    bm = jnp.broadcast_to(
        jnp.concatenate([es2.max(axis=0), ed2.max(axis=0)])[:, None], (2, LANES))

    @pl.when(i == 0)
    def _():
        mx_s[...] = bm

    @pl.when(i > 0)
    def _():
        mx_s[...] = jnp.maximum(mx_s[...], bm)

    m = mx_s[...]
    mo2_ref[...] = _leaky(m[0:1, :] + m[1:2, :])


def _tc2(numa, numb, den1, w2, b2, a2s, a2d):
    grid = (N // _BN,)
    return pl.pallas_call(
        _tc2_body,
        grid=grid,
        in_specs=[
            pl.BlockSpec((NC, _BN, NHID), lambda i: (0, i, 0)),
            pl.BlockSpec((NC, _BN, NHID), lambda i: (0, i, 0)),
            pl.BlockSpec((_BN, NHEADS * NS), lambda i: (i, 0)),
            pl.BlockSpec((NHEADS * NHID, OUTD2), lambda i: (0, 0)),
            pl.BlockSpec((1, OUTD2), lambda i: (0, 0)),
            pl.BlockSpec((OUTD2, 1), lambda i: (0, 0)),
            pl.BlockSpec((OUTD2, 1), lambda i: (0, 0)),
        ],
        out_specs=[
            pl.BlockSpec((_BN, OUTD2), lambda i: (i, 0)),
            pl.BlockSpec((_BN, 1), lambda i: (i, 0)),
            pl.BlockSpec((_BN, 1), lambda i: (i, 0)),
            pl.BlockSpec((1, LANES), lambda i: (0, 0)),
        ],
        out_shape=[
            jax.ShapeDtypeStruct((N, OUTD2), jnp.float32),
            jax.ShapeDtypeStruct((N, 1), jnp.float32),
            jax.ShapeDtypeStruct((N, 1), jnp.float32),
            jax.ShapeDtypeStruct((1, LANES), jnp.float32),
        ],
        scratch_shapes=[pltpu.VMEM((2, LANES), jnp.float32)],
    )(numa, numb, den1, w2, b2, a2s, a2d)


def _tc3_body(num_ref, den_ref, wl_ref, bl_ref, y_ref):
    nm = num_ref[0] + num_ref[1]
    den = jnp.sum(den_ref[...], axis=1)
    o = _elu(nm / (den[:, None] + 1e-16))
    y_ref[...] = jnp.dot(o, wl_ref[...], preferred_element_type=jnp.float32) + bl_ref[...]


def _tc3(num2, den2, wl, bl):
    grid = (N // _BN,)
    return pl.pallas_call(
        _tc3_body,
        grid=grid,
        in_specs=[
            pl.BlockSpec((NC, _BN, OUTD2), lambda i: (0, i, 0)),
            pl.BlockSpec((_BN, NC * NS), lambda i: (i, 0)),
            pl.BlockSpec((OUTD2, 1), lambda i: (0, 0)),
            pl.BlockSpec((1, 1), lambda i: (0, 0)),
        ],
        out_specs=pl.BlockSpec((_BN, 1), lambda i: (i, 0)),
        out_shape=jax.ShapeDtypeStruct((N, 1), jnp.float32),
    )(num2, den2, wl, bl)


def kernel(x, adj, edge_index, W1, a1_src, a1_dst, b1, W2, a2_src, a2_dst, b2, Wl, bl):
    src = edge_index[0]
    dst = edge_index[1]

    w1cat = W1.transpose(1, 0, 2).reshape(NFEAT, NHEADS * NHID)
    b1cat = b1.reshape(1, NHEADS * NHID)
    eye = jnp.eye(NHEADS, dtype=jnp.float32)
    a1s = (a1_src[:, :, None] * eye[:, None, :]).reshape(NHEADS * NHID, NHEADS)
    a1d = (a1_dst[:, :, None] * eye[:, None, :]).reshape(NHEADS * NHID, NHEADS)

    srcp = jnp.concatenate([src, jnp.full((EPTOT - E,), N, jnp.int32)])
    dstp = jnp.concatenate([dst, jnp.full((EPTOT - E,), N, jnp.int32)])

    hh0, hh1, hh2, hh3, es1, ed1, mo1 = _tc1(x, w1cat, b1cat, a1s, a1d)
    hh = [jnp.pad(hx, ((0, NP - N), (0, 0))) for hx in (hh0, hh1, hh2, hh3)]
    e = [jnp.pad(es1[:, h], (0, NP - N)) for h in range(NHEADS)]
    d = [jnp.pad(ed1[:, h], (0, NP - N)) for h in range(NHEADS)]

    ept1 = EPTOT // NS
    numa, dena = _sc_call(ept1, 0, hh[0], hh[1], e[0], e[1], d[0], d[1],
                          mo1[0:2].reshape(NC * LANES), srcp, dstp)
    numb, denb = _sc_call(ept1, 0, hh[2], hh[3], e[2], e[3], d[2], d[3],
                          mo1[2:4].reshape(NC * LANES), srcp, dstp)

    den1t = jnp.concatenate(
        [dena.reshape(NC, NS, NP), denb.reshape(NC, NS, NP)],
        axis=0)[:, :, :N].reshape(NHEADS * NS, N).T

    h2, es2, ed2, mo2 = _tc2(numa[:, :N, :], numb[:, :N, :], den1t,
                             W2, b2.reshape(1, OUTD2),
                             a2_src.reshape(OUTD2, 1), a2_dst.reshape(OUTD2, 1))

    mo2f = jnp.concatenate([mo2, mo2], axis=0).reshape(NC * LANES)
    h2p = jnp.pad(h2, ((0, NP - N), (0, 0)))
    es2f = jnp.pad(es2.reshape(N), (0, NP - N))
    ed2f = jnp.pad(ed2.reshape(N), (0, NP - N))
    ept2 = EPTOT // (NC * NS)
    num2, den2 = _sc_call(ept2, EPTOT // NC, h2p, h2p, es2f, es2f, ed2f, ed2f,
                          mo2f, srcp, dstp)
    den2t = den2.reshape(NC * NS, NP)[:, :N].T

    return _tc3(num2[:, :N, :], den2t, Wl, bl.reshape(1, 1))


# speedup vs baseline: 1.1943x; 1.1943x over previous
import jax
import jax.numpy as jnp
from jax import lax
from jax.experimental import pallas as pl
from jax.experimental.pallas import tpu as pltpu
from jax.experimental.pallas import tpu_sc as plsc

N = 10000
E = 320000
NFEAT = 128
NHID = 64
OUTD2 = 64
NHEADS = 4
ALPHA = 0.2

NC = 2
NS = 16
LANES = 16
FEAT = 64

NP = 10240
CH = 128
EPTOT = 327680
NBUF = 4
GLEAD = 2
NROWS = 10112
ROWS_PT = NROWS // NS
ZR = 128

_BN = 1000


def _leaky(v):
    return jnp.where(v >= 0.0, v, ALPHA * v)


def _elu(v):
    return jnp.where(v > 0.0, v, jnp.exp(v) - 1.0)


def _tc1_body(x_ref, w_ref, b_ref, as_ref, ad_ref,
              h0_ref, h1_ref, h2_ref, h3_ref, es_ref, ed_ref, mo_ref, mx_s):
    i = pl.program_id(0)
    h = jnp.dot(x_ref[...], w_ref[...], preferred_element_type=jnp.float32) + b_ref[...]
    for j, href in enumerate([h0_ref, h1_ref, h2_ref, h3_ref]):
        href[...] = h[:, j * NHID:(j + 1) * NHID]
    es = jnp.dot(h, as_ref[...], preferred_element_type=jnp.float32)
    ed = jnp.dot(h, ad_ref[...], preferred_element_type=jnp.float32)
    es_ref[...] = es
    ed_ref[...] = ed
    bm = jnp.broadcast_to(
        jnp.concatenate([es.max(axis=0), ed.max(axis=0)])[:, None], (8, LANES))

    @pl.when(i == 0)
    def _():
        mx_s[...] = bm

    @pl.when(i > 0)
    def _():
        mx_s[...] = jnp.maximum(mx_s[...], bm)

    m = mx_s[...]
    mo_ref[...] = _leaky(m[:4, :] + m[4:, :])


def _tc1(x, w1cat, b1cat, a1s, a1d):
    grid = (N // _BN,)
    return pl.pallas_call(
        _tc1_body,
        grid=grid,
        in_specs=[
            pl.BlockSpec((_BN, NFEAT), lambda i: (i, 0)),
            pl.BlockSpec((NFEAT, 2 * NFEAT), lambda i: (0, 0)),
            pl.BlockSpec((1, 2 * NFEAT), lambda i: (0, 0)),
            pl.BlockSpec((2 * NFEAT, NHEADS), lambda i: (0, 0)),
            pl.BlockSpec((2 * NFEAT, NHEADS), lambda i: (0, 0)),
        ],
        out_specs=[
            pl.BlockSpec((_BN, NHID), lambda i: (i, 0)),
            pl.BlockSpec((_BN, NHID), lambda i: (i, 0)),
            pl.BlockSpec((_BN, NHID), lambda i: (i, 0)),
            pl.BlockSpec((_BN, NHID), lambda i: (i, 0)),
            pl.BlockSpec((_BN, NHEADS), lambda i: (i, 0)),
            pl.BlockSpec((_BN, NHEADS), lambda i: (i, 0)),
            pl.BlockSpec((NHEADS, LANES), lambda i: (0, 0)),
        ],
        out_shape=[
            jax.ShapeDtypeStruct((N, NHID), jnp.float32),
            jax.ShapeDtypeStruct((N, NHID), jnp.float32),
            jax.ShapeDtypeStruct((N, NHID), jnp.float32),
            jax.ShapeDtypeStruct((N, NHID), jnp.float32),
            jax.ShapeDtypeStruct((N, NHEADS), jnp.float32),
            jax.ShapeDtypeStruct((N, NHEADS), jnp.float32),
            jax.ShapeDtypeStruct((NHEADS, LANES), jnp.float32),
        ],
        scratch_shapes=[pltpu.VMEM((8, LANES), jnp.float32)],
    )(x, w1cat, b1cat, a1s, a1d)


def _make_sc_body(ept, cb):
    nchunk = ept // CH
    assert nchunk % NBUF == 0 and nchunk >= 2 * NBUF

    def body(h_a, h_b, es_a, es_b, ed_a, ed_b, mo, src, dst,
             num_out, den_out,
             es_v, ed_v, mo_v, den_v, exv, srcv, dstv, dsc, rows, zbuf, num_sh,
             isems, gsems, ssems):
        c = lax.axis_index("c")
        s = lax.axis_index("s")

        @pl.when(c == 0)
        def _():
            pltpu.sync_copy(es_a, es_v)
            pltpu.sync_copy(ed_a, ed_v)

        @pl.when(c == 1)
        def _():
            pltpu.sync_copy(es_b, es_v)
            pltpu.sync_copy(ed_b, ed_v)

        pltpu.sync_copy(mo, mo_v)

        def zden(k, carry):
            den_v[pl.ds(k * LANES, LANES)] = jnp.zeros((LANES,), jnp.float32)
            return carry

        lax.fori_loop(0, NP // LANES, zden, 0)

        def zz(k, carry):
            z = jnp.zeros((LANES,), jnp.float32)
            for j in range(FEAT // LANES):
                zbuf[k, pl.ds(j * LANES, LANES)] = z
            return carry

        lax.fori_loop(0, ZR, zz, 0)
        rbase = pl.multiple_of(s * ROWS_PT, 8)
        for t in range(ROWS_PT // ZR):
            pltpu.sync_copy(zbuf, num_sh.at[pl.ds(rbase + t * ZR, ZR)])
        rem = ROWS_PT % ZR
        if rem:
            pltpu.sync_copy(
                zbuf.at[pl.ds(0, rem)],
                num_sh.at[pl.ds(rbase + (ROWS_PT // ZR) * ZR, rem)])
        plsc.subcore_barrier()

        bufs = tuple(
            (exv[u], srcv[u], dstv[u], dsc[u], rows[u], isems[u], gsems[u],
             ssems[u])
            for u in range(NBUF))

        def edge_loop(h_ref):
            moff = mo_v[pl.ds(pl.multiple_of(c * LANES, 8), LANES)]

            def off(b):
                o = c * cb + s * ept + b * CH
                o = jnp.minimum(o, EPTOT - CH)
                return pl.multiple_of(o, 8)

            def issue_idx(b, buf):
                _, sv, dv, _, _, isem, _, _ = bufs[buf]
                o = off(b)
                pltpu.async_copy(src.at[pl.ds(o, CH)], sv, isem)
                pltpu.async_copy(dst.at[pl.ds(o, CH)], dv, isem)

            def wait_idx(buf):
                _, sv, dv, _, _, isem, _, _ = bufs[buf]
                pltpu.make_async_copy(src.at[pl.ds(0, CH)], sv, isem).wait()
                pltpu.make_async_copy(dst.at[pl.ds(0, CH)], dv, isem).wait()

            def issue_gather(buf):
                _, sv, _, _, rw, _, gsem, _ = bufs[buf]
                pltpu.async_copy(h_ref.at[sv], rw, gsem)

            def wait_gather(buf):
                rw, gsem = bufs[buf][4], bufs[buf][6]
                pltpu.make_async_copy(h_ref.at[pl.ds(0, CH)], rw, gsem).wait()

            def wait_scatter(buf):
                rw, ssem = bufs[buf][4], bufs[buf][7]
                pltpu.make_async_copy(h_ref.at[pl.ds(0, CH)], rw, ssem).wait()

            def process(b, buf, skip_ws):
                ex, sv, dv, dc, rw, isem, gsem, ssem = bufs[buf]
                for k in range(CH // LANES):
                    svv = sv[pl.ds(k * LANES, LANES)]
                    dvv = dv[pl.ds(k * LANES, LANES)]
                    a = (plsc.load_gather(es_v, [svv])
                         + plsc.load_gather(ed_v, [dvv]))
                    exl = jnp.exp(_leaky(a) - moff)
                    ex[pl.ds(k * LANES, LANES)] = exl
                    plsc.addupdate_scatter(den_v, [dvv], exl)
                    dc[pl.ds(k * LANES, LANES)] = dvv
                if not skip_ws:
                    wait_scatter((buf + GLEAD) % NBUF)
                wait_idx((buf + GLEAD) % NBUF)
                issue_gather((buf + GLEAD) % NBUF)
                wait_gather(buf)
                issue_idx(b + NBUF, buf)
                def scale(i2, carry2):
                    for u in range(4):
                        i = i2 * 4 + u
                        bidx = jnp.zeros((LANES,), jnp.int32) + i
                        b0 = plsc.load_gather(ex, [bidx])
                        for j in range(FEAT // LANES):
                            sl = rw[i, pl.ds(j * LANES, LANES)]
                            rw[i, pl.ds(j * LANES, LANES)] = sl * b0
                    return carry2

                lax.fori_loop(0, CH // 4, scale, 0)
                pltpu.async_copy(rw, num_sh.at[dc], ssem, add=True)

            for b in range(NBUF):
                issue_idx(b, b)
            for b in range(GLEAD):
                wait_idx(b)
                issue_gather(b)
            for b in range(NBUF):
                process(b, b, skip_ws=(b < GLEAD))

            def bodyn(k, carry):
                for u in range(NBUF):
                    process(NBUF * k + u, u, False)
                return carry

            lax.fori_loop(1, nchunk // NBUF, bodyn, 0)
            for b in range(nchunk + GLEAD, nchunk + NBUF):
                wait_idx(b % NBUF)
            for b in range(nchunk, nchunk + GLEAD):
                wait_gather(b % NBUF)
            for b in range(nchunk - GLEAD, nchunk):
                wait_scatter(b % NBUF)

        @pl.when(c == 0)
        def _():
            edge_loop(h_a)

        @pl.when(c == 1)
        def _():
            edge_loop(h_b)

        plsc.subcore_barrier()
        pltpu.sync_copy(num_sh.at[pl.ds(rbase, ROWS_PT)],
                        num_out.at[c].at[pl.ds(rbase, ROWS_PT)])
        doff = pl.multiple_of((c * NS + s) * NP, 8)
        pltpu.sync_copy(den_v, den_out.at[pl.ds(doff, NP)])

    return body


def _sc_call(ept, cb, h_a, h_b, es_a, es_b, ed_a, ed_b, mo, src, dst):
    mesh = plsc.VectorSubcoreMesh(core_axis_name="c", subcore_axis_name="s",
                                  num_cores=NC, num_subcores=NS)
    f = pl.kernel(
        _make_sc_body(ept, cb),
        out_type=[
            jax.ShapeDtypeStruct((NC, NROWS, FEAT), jnp.float32),
            jax.ShapeDtypeStruct((NC * NS * NP,), jnp.float32),
        ],
        mesh=mesh,
        compiler_params=pltpu.CompilerParams(
            needs_layout_passes=False, use_tc_tiling_on_sc=False),
        scratch_types=[
            pltpu.VMEM((NP,), jnp.float32),
            pltpu.VMEM((NP,), jnp.float32),
            pltpu.VMEM((NC * LANES,), jnp.float32),
            pltpu.VMEM((NP,), jnp.float32),
            tuple(pltpu.VMEM((CH,), jnp.float32) for _ in range(NBUF)),
            tuple(pltpu.VMEM((CH,), jnp.int32) for _ in range(NBUF)),
            tuple(pltpu.VMEM((CH,), jnp.int32) for _ in range(NBUF)),
            tuple(pltpu.VMEM((CH,), jnp.int32) for _ in range(NBUF)),
            tuple(pltpu.VMEM((CH, FEAT), jnp.float32) for _ in range(NBUF)),
            pltpu.VMEM((ZR, FEAT), jnp.float32),
            pltpu.VMEM_SHARED((NROWS, FEAT), jnp.float32),
            tuple(pltpu.SemaphoreType.DMA for _ in range(NBUF)),
            tuple(pltpu.SemaphoreType.DMA for _ in range(NBUF)),
            tuple(pltpu.SemaphoreType.DMA for _ in range(NBUF)),
        ],
    )
    return f(h_a, h_b, es_a, es_b, ed_a, ed_b, mo, src, dst)


def _tc2_body(numa_ref, numb_ref, den_ref, w2_ref, b2_ref, a2s_ref, a2d_ref,
              h2_ref, es2_ref, ed2_ref, mo2_ref, mx_s):
    i = pl.program_id(0)
    acc = jnp.zeros((_BN, OUTD2), jnp.float32)
    for h in range(NHEADS):
        nh = (numa_ref if h < 2 else numb_ref)[h % 2]
        dh = jnp.sum(den_ref[:, h * NS:(h + 1) * NS], axis=1)
        o = _elu(nh / (dh[:, None] + 1e-16))
        acc = acc + jnp.dot(o, w2_ref[h * NHID:(h + 1) * NHID, :],
                            preferred_element_type=jnp.float32)
    h2 = acc + b2_ref[...]
    h2_ref[...] = h2
    es2 = jnp.dot(h2, a2s_ref[...], preferred_element_type=jnp.float32)
    ed2 = jnp.dot(h2, a2d_ref[...], preferred_element_type=jnp.float32)
    es2_ref[...] = es2
    ed2_ref[...] = ed2
    bm = jnp.broadcast_to(
        jnp.concatenate([es2.max(axis=0), ed2.max(axis=0)])[:, None], (2, LANES))

    @pl.when(i == 0)
    def _():
        mx_s[...] = bm

    @pl.when(i > 0)
    def _():
        mx_s[...] = jnp.maximum(mx_s[...], bm)

    m = mx_s[...]
    mo2_ref[...] = _leaky(m[0:1, :] + m[1:2, :])


def _tc2(numa, numb, den1, w2, b2, a2s, a2d):
    grid = (N // _BN,)
    return pl.pallas_call(
        _tc2_body,
        grid=grid,
        in_specs=[
            pl.BlockSpec((NC, _BN, NHID), lambda i: (0, i, 0)),
            pl.BlockSpec((NC, _BN, NHID), lambda i: (0, i, 0)),
            pl.BlockSpec((_BN, NHEADS * NS), lambda i: (i, 0)),
            pl.BlockSpec((NHEADS * NHID, OUTD2), lambda i: (0, 0)),
            pl.BlockSpec((1, OUTD2), lambda i: (0, 0)),
            pl.BlockSpec((OUTD2, 1), lambda i: (0, 0)),
            pl.BlockSpec((OUTD2, 1), lambda i: (0, 0)),
        ],
        out_specs=[
            pl.BlockSpec((_BN, OUTD2), lambda i: (i, 0)),
            pl.BlockSpec((_BN, 1), lambda i: (i, 0)),
            pl.BlockSpec((_BN, 1), lambda i: (i, 0)),
            pl.BlockSpec((1, LANES), lambda i: (0, 0)),
        ],
        out_shape=[
            jax.ShapeDtypeStruct((N, OUTD2), jnp.float32),
            jax.ShapeDtypeStruct((N, 1), jnp.float32),
            jax.ShapeDtypeStruct((N, 1), jnp.float32),
            jax.ShapeDtypeStruct((1, LANES), jnp.float32),
        ],
        scratch_shapes=[pltpu.VMEM((2, LANES), jnp.float32)],
    )(numa, numb, den1, w2, b2, a2s, a2d)


def _tc3_body(num_ref, den_ref, wl_ref, bl_ref, y_ref):
    nm = num_ref[0] + num_ref[1]
    den = jnp.sum(den_ref[...], axis=1)
    o = _elu(nm / (den[:, None] + 1e-16))
    y_ref[...] = jnp.dot(o, wl_ref[...], preferred_element_type=jnp.float32) + bl_ref[...]


def _tc3(num2, den2, wl, bl):
    grid = (N // _BN,)
    return pl.pallas_call(
        _tc3_body,
        grid=grid,
        in_specs=[
            pl.BlockSpec((NC, _BN, OUTD2), lambda i: (0, i, 0)),
            pl.BlockSpec((_BN, NC * NS), lambda i: (i, 0)),
            pl.BlockSpec((OUTD2, 1), lambda i: (0, 0)),
            pl.BlockSpec((1, 1), lambda i: (0, 0)),
        ],
        out_specs=pl.BlockSpec((_BN, 1), lambda i: (i, 0)),
        out_shape=jax.ShapeDtypeStruct((N, 1), jnp.float32),
    )(num2, den2, wl, bl)


def kernel(x, adj, edge_index, W1, a1_src, a1_dst, b1, W2, a2_src, a2_dst, b2, Wl, bl):
    src = edge_index[0]
    dst = edge_index[1]

    w1cat = W1.transpose(1, 0, 2).reshape(NFEAT, NHEADS * NHID)
    b1cat = b1.reshape(1, NHEADS * NHID)
    eye = jnp.eye(NHEADS, dtype=jnp.float32)
    a1s = (a1_src[:, :, None] * eye[:, None, :]).reshape(NHEADS * NHID, NHEADS)
    a1d = (a1_dst[:, :, None] * eye[:, None, :]).reshape(NHEADS * NHID, NHEADS)

    srcp = jnp.concatenate([src, jnp.full((EPTOT - E,), N, jnp.int32)])
    dstp = jnp.concatenate([dst, jnp.full((EPTOT - E,), N, jnp.int32)])

    hh0, hh1, hh2, hh3, es1, ed1, mo1 = _tc1(x, w1cat, b1cat, a1s, a1d)
    hh = [jnp.pad(hx, ((0, NP - N), (0, 0))) for hx in (hh0, hh1, hh2, hh3)]
    e = [jnp.pad(es1[:, h], (0, NP - N)) for h in range(NHEADS)]
    d = [jnp.pad(ed1[:, h], (0, NP - N)) for h in range(NHEADS)]

    ept1 = EPTOT // NS
    numa, dena = _sc_call(ept1, 0, hh[0], hh[1], e[0], e[1], d[0], d[1],
                          mo1[0:2].reshape(NC * LANES), srcp, dstp)
    numb, denb = _sc_call(ept1, 0, hh[2], hh[3], e[2], e[3], d[2], d[3],
                          mo1[2:4].reshape(NC * LANES), srcp, dstp)

    den1t = jnp.concatenate(
        [dena.reshape(NC, NS, NP), denb.reshape(NC, NS, NP)],
        axis=0)[:, :, :N].reshape(NHEADS * NS, N).T

    h2, es2, ed2, mo2 = _tc2(numa[:, :N, :], numb[:, :N, :], den1t,
                             W2, b2.reshape(1, OUTD2),
                             a2_src.reshape(OUTD2, 1), a2_dst.reshape(OUTD2, 1))

    mo2f = jnp.concatenate([mo2, mo2], axis=0).reshape(NC * LANES)
    h2p = jnp.pad(h2, ((0, NP - N), (0, 0)))
    es2f = jnp.pad(es2.reshape(N), (0, NP - N))
    ed2f = jnp.pad(ed2.reshape(N), (0, NP - N))
    ept2 = EPTOT // (NC * NS)
    num2, den2 = _sc_call(ept2, EPTOT // NC, h2p, h2p, es2f, es2f, ed2f, ed2f,
                          mo2f, srcp, dstp)
    den2t = den2.reshape(NC * NS, NP)[:, :N].T

    return _tc3(num2[:, :N, :], den2t, Wl, bl.reshape(1, 1))


# speedup vs baseline: 1.3568x; 1.1361x over previous
import jax
import jax.numpy as jnp
from jax import lax
from jax.experimental import pallas as pl
from jax.experimental.pallas import tpu as pltpu
from jax.experimental.pallas import tpu_sc as plsc

N = 10000
E = 320000
NFEAT = 128
NHID = 64
OUTD2 = 64
NHEADS = 4
ALPHA = 0.2

NC = 2
NS = 16
LANES = 16
FEAT = 64

NP = 10240
CH = 128
EPTOT = 327680
NBUF = 4
GLEAD = 2
NROWS = 10112
ROWS_PT = NROWS // NS
ZR = 128

_BN = 1000


def _leaky(v):
    return jnp.where(v >= 0.0, v, ALPHA * v)


def _elu(v):
    return jnp.where(v > 0.0, v, jnp.exp(v) - 1.0)


def _tc1_body(x_ref, w_ref, b_ref, as_ref, ad_ref,
              h0_ref, h1_ref, h2_ref, h3_ref, es_ref, ed_ref, mo_ref, mx_s):
    i = pl.program_id(0)
    h = jnp.dot(x_ref[...], w_ref[...], preferred_element_type=jnp.float32) + b_ref[...]
    for j, href in enumerate([h0_ref, h1_ref, h2_ref, h3_ref]):
        href[...] = h[:, j * NHID:(j + 1) * NHID]
    es = jnp.dot(h, as_ref[...], preferred_element_type=jnp.float32)
    ed = jnp.dot(h, ad_ref[...], preferred_element_type=jnp.float32)
    es_ref[...] = es
    ed_ref[...] = ed
    bm = jnp.broadcast_to(
        jnp.concatenate([es.max(axis=0), ed.max(axis=0)])[:, None], (8, LANES))

    @pl.when(i == 0)
    def _():
        mx_s[...] = bm

    @pl.when(i > 0)
    def _():
        mx_s[...] = jnp.maximum(mx_s[...], bm)

    m = mx_s[...]
    mo_ref[...] = _leaky(m[:4, :] + m[4:, :])


def _tc1(x, w1cat, b1cat, a1s, a1d):
    grid = (N // _BN,)
    return pl.pallas_call(
        _tc1_body,
        grid=grid,
        in_specs=[
            pl.BlockSpec((_BN, NFEAT), lambda i: (i, 0)),
            pl.BlockSpec((NFEAT, 2 * NFEAT), lambda i: (0, 0)),
            pl.BlockSpec((1, 2 * NFEAT), lambda i: (0, 0)),
            pl.BlockSpec((2 * NFEAT, NHEADS), lambda i: (0, 0)),
            pl.BlockSpec((2 * NFEAT, NHEADS), lambda i: (0, 0)),
        ],
        out_specs=[
            pl.BlockSpec((_BN, NHID), lambda i: (i, 0)),
            pl.BlockSpec((_BN, NHID), lambda i: (i, 0)),
            pl.BlockSpec((_BN, NHID), lambda i: (i, 0)),
            pl.BlockSpec((_BN, NHID), lambda i: (i, 0)),
            pl.BlockSpec((_BN, NHEADS), lambda i: (i, 0)),
            pl.BlockSpec((_BN, NHEADS), lambda i: (i, 0)),
            pl.BlockSpec((NHEADS, LANES), lambda i: (0, 0)),
        ],
        out_shape=[
            jax.ShapeDtypeStruct((N, NHID), jnp.float32),
            jax.ShapeDtypeStruct((N, NHID), jnp.float32),
            jax.ShapeDtypeStruct((N, NHID), jnp.float32),
            jax.ShapeDtypeStruct((N, NHID), jnp.float32),
            jax.ShapeDtypeStruct((N, NHEADS), jnp.float32),
            jax.ShapeDtypeStruct((N, NHEADS), jnp.float32),
            jax.ShapeDtypeStruct((NHEADS, LANES), jnp.float32),
        ],
        scratch_shapes=[pltpu.VMEM((8, LANES), jnp.float32)],
    )(x, w1cat, b1cat, a1s, a1d)


def _make_sc_body(ept, cb):
    nchunk = ept // CH
    assert nchunk % NBUF == 0 and nchunk >= 2 * NBUF

    def body(h_a, h_b, es_a, es_b, ed_a, ed_b, mo, src, dst,
             num_out, den_out,
             es_v, ed_v, mo_v, den_v, exv, srcv, dstv, dsc, rows, rowf, zbuf,
             num_sh, isems, gsems, ssems):
        c = lax.axis_index("c")
        s = lax.axis_index("s")

        @pl.when(c == 0)
        def _():
            pltpu.sync_copy(es_a, es_v)
            pltpu.sync_copy(ed_a, ed_v)

        @pl.when(c == 1)
        def _():
            pltpu.sync_copy(es_b, es_v)
            pltpu.sync_copy(ed_b, ed_v)

        pltpu.sync_copy(mo, mo_v)

        def zden(k, carry):
            den_v[pl.ds(k * LANES, LANES)] = jnp.zeros((LANES,), jnp.float32)
            return carry

        lax.fori_loop(0, NP // LANES, zden, 0)

        def zz(k, carry):
            z = jnp.zeros((LANES,), jnp.float32)
            for j in range(FEAT // LANES):
                zbuf[k, pl.ds(j * LANES, LANES)] = z
            return carry

        lax.fori_loop(0, ZR, zz, 0)
        rbase = pl.multiple_of(s * ROWS_PT, 8)
        for t in range(ROWS_PT // ZR):
            pltpu.sync_copy(zbuf, num_sh.at[pl.ds(rbase + t * ZR, ZR)])
        rem = ROWS_PT % ZR
        if rem:
            pltpu.sync_copy(
                zbuf.at[pl.ds(0, rem)],
                num_sh.at[pl.ds(rbase + (ROWS_PT // ZR) * ZR, rem)])
        plsc.subcore_barrier()

        bufs = tuple(
            (exv[u], srcv[u], dstv[u], dsc[u], rows[u], rowf[u], isems[u],
             gsems[u], ssems[u])
            for u in range(NBUF))

        def edge_loop(h_ref):
            moff = mo_v[pl.ds(pl.multiple_of(c * LANES, 8), LANES)]

            def off(b):
                o = c * cb + s * ept + b * CH
                o = jnp.minimum(o, EPTOT - CH)
                return pl.multiple_of(o, 8)

            def issue_idx(b, buf):
                _, sv, dv, _, _, _, isem, _, _ = bufs[buf]
                o = off(b)
                pltpu.async_copy(src.at[pl.ds(o, CH)], sv, isem)
                pltpu.async_copy(dst.at[pl.ds(o, CH)], dv, isem)

            def wait_idx(buf):
                _, sv, dv, _, _, _, isem, _, _ = bufs[buf]
                pltpu.make_async_copy(src.at[pl.ds(0, CH)], sv, isem).wait()
                pltpu.make_async_copy(dst.at[pl.ds(0, CH)], dv, isem).wait()

            def issue_gather(buf):
                _, sv, _, _, rw, _, _, gsem, _ = bufs[buf]
                pltpu.async_copy(h_ref.at[sv], rw, gsem)

            def wait_gather(buf):
                rw, gsem = bufs[buf][4], bufs[buf][7]
                pltpu.make_async_copy(h_ref.at[pl.ds(0, CH)], rw, gsem).wait()

            def wait_scatter(buf):
                rf, dc, ssem = bufs[buf][5], bufs[buf][3], bufs[buf][8]
                pltpu.make_async_copy(rf, num_sh.at[dc], ssem).wait()

            def process(b, buf, skip_ws):
                ex, sv, dv, dc, rw, rf, isem, gsem, ssem = bufs[buf]
                for k in range(CH // LANES):
                    svv = sv[pl.ds(k * LANES, LANES)]
                    dvv = dv[pl.ds(k * LANES, LANES)]
                    a = (plsc.load_gather(es_v, [svv])
                         + plsc.load_gather(ed_v, [dvv]))
                    exl = jnp.exp(_leaky(a) - moff)
                    ex[pl.ds(k * LANES, LANES)] = exl
                    plsc.addupdate_scatter(den_v, [dvv], exl)
                    dc[pl.ds(k * LANES, LANES)] = dvv
                if not skip_ws:
                    wait_scatter((buf + GLEAD) % NBUF)
                wait_idx((buf + GLEAD) % NBUF)
                issue_gather((buf + GLEAD) % NBUF)
                wait_gather(buf)
                issue_idx(b + NBUF, buf)
                def scale(i2, carry2):
                    for u in range(2):
                        i = i2 * 2 + u
                        bidx = jnp.zeros((LANES,), jnp.int32) + i
                        b0 = plsc.load_gather(ex, [bidx])
                        for j in range(FEAT // 32):
                            sl = rw[i, pl.ds(j * 32, 32)]
                            u0, u1 = plsc.unpack(
                                sl, format=plsc.PackFormat.INTERLEAVED)
                            rf[i, pl.ds(j * 32, LANES)] = u0 * b0
                            rf[i, pl.ds(j * 32 + LANES, LANES)] = u1 * b0
                    return carry2

                lax.fori_loop(0, CH // 2, scale, 0)
                pltpu.async_copy(rf, num_sh.at[dc], ssem, add=True)

            for b in range(NBUF):
                issue_idx(b, b)
            for b in range(GLEAD):
                wait_idx(b)
                issue_gather(b)
            for b in range(NBUF):
                process(b, b, skip_ws=(b < GLEAD))

            def bodyn(k, carry):
                for u in range(NBUF):
                    process(NBUF * k + u, u, False)
                return carry

            lax.fori_loop(1, nchunk // NBUF, bodyn, 0)
            for b in range(nchunk + GLEAD, nchunk + NBUF):
                wait_idx(b % NBUF)
            for b in range(nchunk, nchunk + GLEAD):
                wait_gather(b % NBUF)
            for b in range(nchunk - GLEAD, nchunk):
                wait_scatter(b % NBUF)

        @pl.when(c == 0)
        def _():
            edge_loop(h_a)

        @pl.when(c == 1)
        def _():
            edge_loop(h_b)

        plsc.subcore_barrier()
        pltpu.sync_copy(num_sh.at[pl.ds(rbase, ROWS_PT)],
                        num_out.at[c].at[pl.ds(rbase, ROWS_PT)])
        doff = pl.multiple_of((c * NS + s) * NP, 8)
        pltpu.sync_copy(den_v, den_out.at[pl.ds(doff, NP)])

    return body


def _sc_call(ept, cb, h_a, h_b, es_a, es_b, ed_a, ed_b, mo, src, dst):
    mesh = plsc.VectorSubcoreMesh(core_axis_name="c", subcore_axis_name="s",
                                  num_cores=NC, num_subcores=NS)
    f = pl.kernel(
        _make_sc_body(ept, cb),
        out_type=[
            jax.ShapeDtypeStruct((NC, NROWS, FEAT), jnp.float32),
            jax.ShapeDtypeStruct((NC * NS * NP,), jnp.float32),
        ],
        mesh=mesh,
        compiler_params=pltpu.CompilerParams(
            needs_layout_passes=False, use_tc_tiling_on_sc=False),
        scratch_types=[
            pltpu.VMEM((NP,), jnp.float32),
            pltpu.VMEM((NP,), jnp.float32),
            pltpu.VMEM((NC * LANES,), jnp.float32),
            pltpu.VMEM((NP,), jnp.float32),
            tuple(pltpu.VMEM((CH,), jnp.float32) for _ in range(NBUF)),
            tuple(pltpu.VMEM((CH,), jnp.int32) for _ in range(NBUF)),
            tuple(pltpu.VMEM((CH,), jnp.int32) for _ in range(NBUF)),
            tuple(pltpu.VMEM((CH,), jnp.int32) for _ in range(NBUF)),
            tuple(pltpu.VMEM((CH, FEAT), jnp.bfloat16) for _ in range(NBUF)),
            tuple(pltpu.VMEM((CH, FEAT), jnp.float32) for _ in range(NBUF)),
            pltpu.VMEM((ZR, FEAT), jnp.float32),
            pltpu.VMEM_SHARED((NROWS, FEAT), jnp.float32),
            tuple(pltpu.SemaphoreType.DMA for _ in range(NBUF)),
            tuple(pltpu.SemaphoreType.DMA for _ in range(NBUF)),
            tuple(pltpu.SemaphoreType.DMA for _ in range(NBUF)),
        ],
    )
    return f(h_a, h_b, es_a, es_b, ed_a, ed_b, mo, src, dst)


def _tc2_body(numa_ref, numb_ref, den_ref, w2_ref, b2_ref, a2s_ref, a2d_ref,
              h2_ref, es2_ref, ed2_ref, mo2_ref, mx_s):
    i = pl.program_id(0)
    acc = jnp.zeros((_BN, OUTD2), jnp.float32)
    for h in range(NHEADS):
        nh = (numa_ref if h < 2 else numb_ref)[h % 2]
        dh = jnp.sum(den_ref[:, h * NS:(h + 1) * NS], axis=1)
        o = _elu(nh / (dh[:, None] + 1e-16))
        acc = acc + jnp.dot(o, w2_ref[h * NHID:(h + 1) * NHID, :],
                            preferred_element_type=jnp.float32)
    h2 = acc + b2_ref[...]
    h2_ref[...] = h2
    es2 = jnp.dot(h2, a2s_ref[...], preferred_element_type=jnp.float32)
    ed2 = jnp.dot(h2, a2d_ref[...], preferred_element_type=jnp.float32)
    es2_ref[...] = es2
    ed2_ref[...] = ed2
    bm = jnp.broadcast_to(
        jnp.concatenate([es2.max(axis=0), ed2.max(axis=0)])[:, None], (2, LANES))

    @pl.when(i == 0)
    def _():
        mx_s[...] = bm

    @pl.when(i > 0)
    def _():
        mx_s[...] = jnp.maximum(mx_s[...], bm)

    m = mx_s[...]
    mo2_ref[...] = _leaky(m[0:1, :] + m[1:2, :])


def _tc2(numa, numb, den1, w2, b2, a2s, a2d):
    grid = (N // _BN,)
    return pl.pallas_call(
        _tc2_body,
        grid=grid,
        in_specs=[
            pl.BlockSpec((NC, _BN, NHID), lambda i: (0, i, 0)),
            pl.BlockSpec((NC, _BN, NHID), lambda i: (0, i, 0)),
            pl.BlockSpec((_BN, NHEADS * NS), lambda i: (i, 0)),
            pl.BlockSpec((NHEADS * NHID, OUTD2), lambda i: (0, 0)),
            pl.BlockSpec((1, OUTD2), lambda i: (0, 0)),
            pl.BlockSpec((OUTD2, 1), lambda i: (0, 0)),
            pl.BlockSpec((OUTD2, 1), lambda i: (0, 0)),
        ],
        out_specs=[
            pl.BlockSpec((_BN, OUTD2), lambda i: (i, 0)),
            pl.BlockSpec((_BN, 1), lambda i: (i, 0)),
            pl.BlockSpec((_BN, 1), lambda i: (i, 0)),
            pl.BlockSpec((1, LANES), lambda i: (0, 0)),
        ],
        out_shape=[
            jax.ShapeDtypeStruct((N, OUTD2), jnp.float32),
            jax.ShapeDtypeStruct((N, 1), jnp.float32),
            jax.ShapeDtypeStruct((N, 1), jnp.float32),
            jax.ShapeDtypeStruct((1, LANES), jnp.float32),
        ],
        scratch_shapes=[pltpu.VMEM((2, LANES), jnp.float32)],
    )(numa, numb, den1, w2, b2, a2s, a2d)


def _tc3_body(num_ref, den_ref, wl_ref, bl_ref, y_ref):
    nm = num_ref[0] + num_ref[1]
    den = jnp.sum(den_ref[...], axis=1)
    o = _elu(nm / (den[:, None] + 1e-16))
    y_ref[...] = jnp.dot(o, wl_ref[...], preferred_element_type=jnp.float32) + bl_ref[...]


def _tc3(num2, den2, wl, bl):
    grid = (N // _BN,)
    return pl.pallas_call(
        _tc3_body,
        grid=grid,
        in_specs=[
            pl.BlockSpec((NC, _BN, OUTD2), lambda i: (0, i, 0)),
            pl.BlockSpec((_BN, NC * NS), lambda i: (i, 0)),
            pl.BlockSpec((OUTD2, 1), lambda i: (0, 0)),
            pl.BlockSpec((1, 1), lambda i: (0, 0)),
        ],
        out_specs=pl.BlockSpec((_BN, 1), lambda i: (i, 0)),
        out_shape=jax.ShapeDtypeStruct((N, 1), jnp.float32),
    )(num2, den2, wl, bl)


def _unsplit(a):
    g = a.reshape(NC, NROWS, FEAT // 32, 2, LANES)
    return g.transpose(0, 1, 2, 4, 3).reshape(NC, NROWS, FEAT)


def kernel(x, adj, edge_index, W1, a1_src, a1_dst, b1, W2, a2_src, a2_dst, b2, Wl, bl):
    src = edge_index[0]
    dst = edge_index[1]

    w1cat = W1.transpose(1, 0, 2).reshape(NFEAT, NHEADS * NHID)
    b1cat = b1.reshape(1, NHEADS * NHID)
    eye = jnp.eye(NHEADS, dtype=jnp.float32)
    a1s = (a1_src[:, :, None] * eye[:, None, :]).reshape(NHEADS * NHID, NHEADS)
    a1d = (a1_dst[:, :, None] * eye[:, None, :]).reshape(NHEADS * NHID, NHEADS)

    srcp = jnp.concatenate([src, jnp.full((EPTOT - E,), N, jnp.int32)])
    dstp = jnp.concatenate([dst, jnp.full((EPTOT - E,), N, jnp.int32)])

    hh0, hh1, hh2, hh3, es1, ed1, mo1 = _tc1(x, w1cat, b1cat, a1s, a1d)
    hh = [jnp.pad(hx.astype(jnp.bfloat16), ((0, NP - N), (0, 0)))
          for hx in (hh0, hh1, hh2, hh3)]
    e = [jnp.pad(es1[:, h], (0, NP - N)) for h in range(NHEADS)]
    d = [jnp.pad(ed1[:, h], (0, NP - N)) for h in range(NHEADS)]

    ept1 = EPTOT // NS
    numa, dena = _sc_call(ept1, 0, hh[0], hh[1], e[0], e[1], d[0], d[1],
                          mo1[0:2].reshape(NC * LANES), srcp, dstp)
    numb, denb = _sc_call(ept1, 0, hh[2], hh[3], e[2], e[3], d[2], d[3],
                          mo1[2:4].reshape(NC * LANES), srcp, dstp)

    den1t = jnp.concatenate(
        [dena.reshape(NC, NS, NP), denb.reshape(NC, NS, NP)],
        axis=0)[:, :, :N].reshape(NHEADS * NS, N).T

    h2, es2, ed2, mo2 = _tc2(_unsplit(numa)[:, :N, :], _unsplit(numb)[:, :N, :], den1t,
                             W2, b2.reshape(1, OUTD2),
                             a2_src.reshape(OUTD2, 1), a2_dst.reshape(OUTD2, 1))

    mo2f = jnp.concatenate([mo2, mo2], axis=0).reshape(NC * LANES)
    h2p = jnp.pad(h2.astype(jnp.bfloat16), ((0, NP - N), (0, 0)))
    es2f = jnp.pad(es2.reshape(N), (0, NP - N))
    ed2f = jnp.pad(ed2.reshape(N), (0, NP - N))
    ept2 = EPTOT // (NC * NS)
    num2, den2 = _sc_call(ept2, EPTOT // NC, h2p, h2p, es2f, es2f, ed2f, ed2f,
                          mo2f, srcp, dstp)
    den2t = den2.reshape(NC * NS, NP)[:, :N].T

    return _tc3(_unsplit(num2)[:, :N, :], den2t, Wl, bl.reshape(1, 1))
